# Initial kernel scaffold; baseline (speedup 1.0000x reference)
import functools

import jax
import jax.numpy as jnp
from jax import lax
from jax.experimental import pallas as pl
from jax.experimental.pallas import tpu as pltpu
from jax.experimental.pallas import tpu_sc as plsc

N_AXES = 4
PER_AXIS = 32
TABLE_ROWS = 4096
NUM_WORKERS = 32
IDS_PER_ROW = 128
ROWS_PER_WORKER = 32
CHUNK_ROWS = 8
N_CHUNKS = ROWS_PER_WORKER // CHUNK_ROWS
CHUNK_IDS = CHUNK_ROWS * IDS_PER_ROW
NBUF = 2


def _embed_body(ids_hbm, w_hbm, out_hbm, idx_v, rows_v, sems):
    wid = lax.axis_index("s") * 2 + lax.axis_index("c")
    axis_off = (lax.iota(jnp.int32, (16,)) % 4) * TABLE_ROWS

    def load_chunk(c, buf):
        pltpu.sync_copy(ids_hbm.at[wid, pl.ds(c * CHUNK_ROWS, CHUNK_ROWS)],
                        idx_v.at[buf])
        for j in range(CHUNK_ROWS):
            for k in range(IDS_PER_ROW // 16):
                sl = pl.ds(k * 16, 16)
                idx_v[buf, j, sl] = idx_v[buf, j, sl] + axis_off
        for j in range(CHUNK_ROWS):
            pltpu.make_async_copy(
                w_hbm.at[idx_v.at[buf, j]],
                rows_v.at[buf, pl.ds(j * IDS_PER_ROW, IDS_PER_ROW)],
                sems.at[buf],
            ).start()

    def drain_chunk(c, buf):
        for j in range(CHUNK_ROWS):
            pltpu.make_async_copy(
                w_hbm.at[idx_v.at[buf, j]],
                rows_v.at[buf, pl.ds(j * IDS_PER_ROW, IDS_PER_ROW)],
                sems.at[buf],
            ).wait()
        base = wid * ROWS_PER_WORKER * IDS_PER_ROW + c * CHUNK_IDS
        pltpu.sync_copy(rows_v.at[buf], out_hbm.at[pl.ds(base, CHUNK_IDS)])

    load_chunk(0, 0)
    for c in range(N_CHUNKS):
        if c + 1 < N_CHUNKS:
            load_chunk(c + 1, (c + 1) % NBUF)
        drain_chunk(c, c % NBUF)


def kernel(ids, W0, W1, W2, W3):
    batch, seq, n_axes = ids.shape
    n_ids = batch * seq * n_axes
    ids_r = ids.astype(jnp.int32).reshape(
        NUM_WORKERS, ROWS_PER_WORKER, IDS_PER_ROW)
    w_cat = jnp.concatenate([W0, W1, W2, W3], axis=0)

    mesh = plsc.VectorSubcoreMesh(core_axis_name="c", subcore_axis_name="s")
    run = functools.partial(
        pl.kernel,
        out_type=jax.ShapeDtypeStruct((n_ids, PER_AXIS), jnp.float32),
        mesh=mesh,
        scratch_types=[
            pltpu.VMEM((NBUF, CHUNK_ROWS, IDS_PER_ROW), jnp.int32),
            pltpu.VMEM((NBUF, CHUNK_IDS, PER_AXIS), jnp.float32),
            pltpu.SemaphoreType.DMA((NBUF,)),
        ],
    )(_embed_body)
    out = run(ids_r, w_cat)
    return out.reshape(batch, 1, seq, n_axes * PER_AXIS)


# speedup vs baseline: 7.8604x; 7.8604x over previous
import functools

import jax
import jax.numpy as jnp
from jax import lax
from jax.experimental import pallas as pl
from jax.experimental.pallas import tpu as pltpu
from jax.experimental.pallas import tpu_sc as plsc

N_AXES = 4
PER_AXIS = 32
TABLE_ROWS = 4096
NUM_WORKERS = 32
IDS_PER_ROW = 128
ROWS_PER_WORKER = 32
CHUNK_ROWS = 8
N_CHUNKS = ROWS_PER_WORKER // CHUNK_ROWS
CHUNK_IDS = CHUNK_ROWS * IDS_PER_ROW
NBUF = 2


def _embed_body(ids_hbm, w_hbm, out_hbm, idx_v, rows_v, sems):
    wid = lax.axis_index("s") * 2 + lax.axis_index("c")
    axis_off = (lax.iota(jnp.int32, 16) % 4) * TABLE_ROWS

    def load_chunk(c, buf):
        pltpu.sync_copy(ids_hbm.at[wid, pl.ds(c * CHUNK_ROWS, CHUNK_ROWS)],
                        idx_v.at[buf])
        for j in range(CHUNK_ROWS):
            for k in range(IDS_PER_ROW // 16):
                sl = pl.ds(k * 16, 16)
                idx_v[buf, j, sl] = idx_v[buf, j, sl] + axis_off
        for j in range(CHUNK_ROWS):
            pltpu.make_async_copy(
                w_hbm.at[idx_v.at[buf, j]],
                rows_v.at[buf, pl.ds(j * IDS_PER_ROW, IDS_PER_ROW)],
                sems.at[buf],
            ).start()

    def drain_chunk(c, buf):
        for j in range(CHUNK_ROWS):
            pltpu.make_async_copy(
                w_hbm.at[idx_v.at[buf, j]],
                rows_v.at[buf, pl.ds(j * IDS_PER_ROW, IDS_PER_ROW)],
                sems.at[buf],
            ).wait()
        base = wid * ROWS_PER_WORKER * IDS_PER_ROW + c * CHUNK_IDS
        pltpu.sync_copy(rows_v.at[buf], out_hbm.at[pl.ds(base, CHUNK_IDS)])

    load_chunk(0, 0)
    for c in range(N_CHUNKS):
        if c + 1 < N_CHUNKS:
            load_chunk(c + 1, (c + 1) % NBUF)
        drain_chunk(c, c % NBUF)


def kernel(ids, W0, W1, W2, W3):
    batch, seq, n_axes = ids.shape
    n_ids = batch * seq * n_axes
    ids_r = ids.astype(jnp.int32).reshape(
        NUM_WORKERS, ROWS_PER_WORKER, IDS_PER_ROW)
    w_cat = jnp.concatenate([W0, W1, W2, W3], axis=0)

    mesh = plsc.VectorSubcoreMesh(core_axis_name="c", subcore_axis_name="s")
    run = functools.partial(
        pl.kernel,
        out_type=jax.ShapeDtypeStruct((n_ids, PER_AXIS), jnp.float32),
        mesh=mesh,
        scratch_types=[
            pltpu.VMEM((NBUF, CHUNK_ROWS, IDS_PER_ROW), jnp.int32),
            pltpu.VMEM((NBUF, CHUNK_IDS, PER_AXIS), jnp.float32),
            pltpu.SemaphoreType.DMA((NBUF,)),
        ],
        compiler_params=pltpu.CompilerParams(use_tc_tiling_on_sc=False),
    )(_embed_body)
    out = run(ids_r, w_cat)
    return out.reshape(batch, 1, seq, n_axes * PER_AXIS)


# speedup vs baseline: 8.2397x; 1.0483x over previous
import functools

import jax
import jax.numpy as jnp
from jax import lax
from jax.experimental import pallas as pl
from jax.experimental.pallas import tpu as pltpu
from jax.experimental.pallas import tpu_sc as plsc

N_AXES = 4
PER_AXIS = 32
NUM_WORKERS = 32
TOK_PER_WORKER = 1024
CHUNK_TOK = 256
N_CHUNKS = TOK_PER_WORKER // CHUNK_TOK
IDS_PER_ROW = 128
ROWS_PER_AXIS = CHUNK_TOK // IDS_PER_ROW
NBUF = 2


def _embed_body(ids_hbm, w0, w1, w2, w3, out_hbm, raw_v, idx_v, rows_v, sems):
    tables = (w0, w1, w2, w3)
    wid = lax.axis_index("s") * 2 + lax.axis_index("c")
    tok0_w = wid * TOK_PER_WORKER
    lane = lax.iota(jnp.int32, 16)

    def load_chunk(c, buf):
        pltpu.sync_copy(
            ids_hbm.at[pl.ds((tok0_w + c * CHUNK_TOK) * N_AXES,
                             CHUNK_TOK * N_AXES)],
            raw_v.at[buf])
        for i in range(N_AXES):
            for g in range(CHUNK_TOK // 16):
                v = plsc.load_gather(raw_v.at[buf],
                                     [lane * N_AXES + (g * 16 * N_AXES + i)])
                j, col = (g * 16) // IDS_PER_ROW, (g * 16) % IDS_PER_ROW
                idx_v[buf, i, j, pl.ds(col, 16)] = v
        for i in range(N_AXES):
            for j in range(ROWS_PER_AXIS):
                pltpu.make_async_copy(
                    tables[i].at[idx_v.at[buf, i, j]],
                    rows_v.at[buf, i, pl.ds(j * IDS_PER_ROW, IDS_PER_ROW)],
                    sems.at[buf],
                ).start()

    def drain_chunk(c, buf):
        for i in range(N_AXES):
            for j in range(ROWS_PER_AXIS):
                pltpu.make_async_copy(
                    tables[i].at[idx_v.at[buf, i, j]],
                    rows_v.at[buf, i, pl.ds(j * IDS_PER_ROW, IDS_PER_ROW)],
                    sems.at[buf],
                ).wait()
        tok0 = tok0_w + c * CHUNK_TOK
        for i in range(N_AXES):
            pltpu.sync_copy(
                rows_v.at[buf, i],
                out_hbm.at[pl.ds(tok0, CHUNK_TOK),
                           pl.ds(i * PER_AXIS, PER_AXIS)])

    load_chunk(0, 0)
    for c in range(N_CHUNKS):
        if c + 1 < N_CHUNKS:
            load_chunk(c + 1, (c + 1) % NBUF)
        drain_chunk(c, c % NBUF)


def kernel(ids, W0, W1, W2, W3):
    batch, seq, n_axes = ids.shape
    n_tok = batch * seq

    mesh = plsc.VectorSubcoreMesh(core_axis_name="c", subcore_axis_name="s")
    run = functools.partial(
        pl.kernel,
        out_type=jax.ShapeDtypeStruct((n_tok, N_AXES * PER_AXIS), jnp.float32),
        mesh=mesh,
        scratch_types=[
            pltpu.VMEM((NBUF, CHUNK_TOK * N_AXES), jnp.int32),
            pltpu.VMEM((NBUF, N_AXES, ROWS_PER_AXIS, IDS_PER_ROW), jnp.int32),
            pltpu.VMEM((NBUF, N_AXES, CHUNK_TOK, PER_AXIS), jnp.float32),
            pltpu.SemaphoreType.DMA((NBUF,)),
        ],
        compiler_params=pltpu.CompilerParams(
            use_tc_tiling_on_sc=False, needs_layout_passes=False),
    )(_embed_body)
    out = run(ids.astype(jnp.int32).reshape(-1), W0, W1, W2, W3)
    return out.reshape(batch, 1, seq, N_AXES * PER_AXIS)


# speedup vs baseline: 8.2555x; 1.0019x over previous
import functools

import jax
import jax.numpy as jnp
from jax import lax
from jax.experimental import pallas as pl
from jax.experimental.pallas import tpu as pltpu
from jax.experimental.pallas import tpu_sc as plsc

N_AXES = 4
PER_AXIS = 32
NUM_WORKERS = 32
TOK_PER_WORKER = 1024
CHUNK_TOK = 256
N_CHUNKS = TOK_PER_WORKER // CHUNK_TOK
IDS_PER_ROW = 128
ROWS_PER_AXIS = CHUNK_TOK // IDS_PER_ROW
NBUF = 2


def _embed_body(ids_hbm, w0, w1, w2, w3, out_hbm, raw_v, idx_v, rows_v, sems):
    tables = (w0, w1, w2, w3)
    wid = lax.axis_index("s") * 2 + lax.axis_index("c")
    tok0_w = wid * TOK_PER_WORKER
    lane = lax.iota(jnp.int32, 16)

    def load_chunk(c, buf):
        pltpu.sync_copy(
            ids_hbm.at[pl.ds((tok0_w + c * CHUNK_TOK) * N_AXES,
                             CHUNK_TOK * N_AXES)],
            raw_v.at[buf])
        for i in range(N_AXES):
            for g in range(CHUNK_TOK // 16):
                v = plsc.load_gather(raw_v.at[buf],
                                     [lane * N_AXES + (g * 16 * N_AXES + i)])
                j, col = (g * 16) // IDS_PER_ROW, (g * 16) % IDS_PER_ROW
                idx_v[buf, i, j, pl.ds(col, 16)] = v
        for i in range(N_AXES):
            for j in range(ROWS_PER_AXIS):
                pltpu.make_async_copy(
                    tables[i].at[idx_v.at[buf, i, j]],
                    rows_v.at[buf, i, pl.ds(j * IDS_PER_ROW, IDS_PER_ROW)],
                    sems.at[buf],
                ).start()

    def drain_chunk(c, buf):
        for i in range(N_AXES):
            for j in range(ROWS_PER_AXIS):
                pltpu.make_async_copy(
                    tables[i].at[idx_v.at[buf, i, j]],
                    rows_v.at[buf, i, pl.ds(j * IDS_PER_ROW, IDS_PER_ROW)],
                    sems.at[buf],
                ).wait()
        tok0 = tok0_w + c * CHUNK_TOK
        seq = out_hbm.shape[2]
        for i in range(N_AXES):
            pltpu.sync_copy(
                rows_v.at[buf, i],
                out_hbm.at[tok0 // seq, 0, pl.ds(tok0 % seq, CHUNK_TOK),
                           pl.ds(i * PER_AXIS, PER_AXIS)])

    load_chunk(0, 0)
    for c in range(N_CHUNKS):
        if c + 1 < N_CHUNKS:
            load_chunk(c + 1, (c + 1) % NBUF)
        drain_chunk(c, c % NBUF)


def kernel(ids, W0, W1, W2, W3):
    batch, seq, n_axes = ids.shape
    n_tok = batch * seq

    mesh = plsc.VectorSubcoreMesh(core_axis_name="c", subcore_axis_name="s")
    run = functools.partial(
        pl.kernel,
        out_type=jax.ShapeDtypeStruct((batch, 1, seq, N_AXES * PER_AXIS),
                                      jnp.float32),
        mesh=mesh,
        scratch_types=[
            pltpu.VMEM((NBUF, CHUNK_TOK * N_AXES), jnp.int32),
            pltpu.VMEM((NBUF, N_AXES, ROWS_PER_AXIS, IDS_PER_ROW), jnp.int32),
            pltpu.VMEM((NBUF, N_AXES, CHUNK_TOK, PER_AXIS), jnp.float32),
            pltpu.SemaphoreType.DMA((NBUF,)),
        ],
        compiler_params=pltpu.CompilerParams(
            use_tc_tiling_on_sc=False, needs_layout_passes=False),
    )(_embed_body)
    return run(ids.astype(jnp.int32).reshape(-1), W0, W1, W2, W3)
